# baseline (speedup 1.0000x reference)
import functools

import jax
import jax.numpy as jnp
from jax import lax
from jax.experimental import pallas as pl
from jax.experimental.pallas import tpu as pltpu
from jax.experimental.pallas import tpu_sc as plsc

S = 7
NB = 2
C = 80
W = NB * 5 + C
NCELL = S * S
BATCH = 128
NGT = 24
L_COORD = 5.0
L_NOOBJ = 0.5

PW = NCELL * W
BL = 160
NC, NS = 2, 16
NTILES = NC * NS
IPT = BATCH // NTILES
PBLK = IPT * PW
CHUNKS = [(0, 4416), (4408, 4416), (8816, 4416), (13224, 4416)]


def _rsqrt_nr(a):
    i = plsc.bitcast(a, jnp.int32)
    y = plsc.bitcast(jnp.int32(0x5F3759DF) - lax.shift_right_arithmetic(i, 1),
                     jnp.float32)
    for _ in range(3):
        y = y * (1.5 - 0.5 * a * y * y)
    return y


@functools.partial(
    pl.kernel,
    out_type=jax.ShapeDtypeStruct((NTILES, 64), jnp.float32),
    mesh=plsc.VectorSubcoreMesh(core_axis_name="c", subcore_axis_name="s",
                                num_cores=NC, num_subcores=NS),
    scratch_types=(
        [pltpu.VMEM((PBLK,), jnp.float32)]
        + [pltpu.VMEM((BL,), jnp.float32) for _ in range(IPT)]
        + [pltpu.VMEM((64,), jnp.int32),
           pltpu.VMEM((64,), jnp.float32)]
        + [pltpu.SemaphoreType.DMA for _ in range(IPT)]
    ),
    compiler_params=pltpu.CompilerParams(needs_layout_passes=False),
)
def _yolo_sc(preds_hbm, bl_hbm, out_hbm, *refs):
    pvb = refs[0]
    bls = refs[1:1 + IPT]
    kb = refs[1 + IPT]
    ov = refs[2 + IPT]
    sems = refs[3 + IPT:]

    wid = lax.axis_index("s") * NC + lax.axis_index("c")
    iota = lax.broadcasted_iota(jnp.int32, (16,), 0)
    zf = jnp.zeros((16,), jnp.float32)

    copies = []
    for t in range(IPT):
        img = wid * IPT + t
        st, ln = CHUNKS[t]
        src_off = pl.multiple_of(wid * PBLK + st, 8)
        c1 = pltpu.async_copy(preds_hbm.at[pl.ds(src_off, ln)],
                              pvb.at[pl.ds(st, ln)], sems[t])
        c2 = pltpu.async_copy(bl_hbm.at[img], bls[t], sems[t])
        copies.append((c1, c2))

    for wdx in range(4):
        kb[pl.ds(wdx * 16, 16)] = -(iota + (wdx * 16 + 100))

    a_coord = a_obj = a_mconf = a_cls = a_dense = zf

    for t in range(IPT):
        for c in copies[t]:
            c.wait()
        pv = pvb
        pbase = t * PW
        bv = bls[t]

        for gs in range(7):
            slot = iota + gs * 16
            off = (slot // NB) * W + (slot % NB) * 5 + (pbase + 4)
            v = plsc.load_gather(pv, [off])
            if gs == 6:
                v = jnp.where(slot < NCELL * NB, v, 0.0)
            a_dense = a_dense + v * v

        gdata = []
        for g in range(2):
            o = g * 16
            x1 = bv[pl.ds(0 + o, 16)]
            y1 = bv[pl.ds(32 + o, 16)]
            x2 = bv[pl.ds(64 + o, 16)]
            y2 = bv[pl.ds(96 + o, 16)]
            lab = plsc.bitcast(bv[pl.ds(128 + o, 16)], jnp.int32)
            valid = (iota + o) < NGT
            gcx = (x1 + x2) * 0.5
            gcy = (y1 + y2) * 0.5
            gw = x2 - x1
            gh = y2 - y1
            gi = jnp.clip((gcx * S).astype(jnp.int32), 0, S - 1)
            gj = jnp.clip((gcy * S).astype(jnp.int32), 0, S - 1)
            gif = gi.astype(jnp.float32)
            gjf = gj.astype(jnp.float32)
            cell = gj * S + gi
            base = cell * W + pbase

            ious = []
            pb = []
            for n in range(NB):
                px = plsc.load_gather(pv, [base + (5 * n + 0)])
                py = plsc.load_gather(pv, [base + (5 * n + 1)])
                pw = plsc.load_gather(pv, [base + (5 * n + 2)])
                ph = plsc.load_gather(pv, [base + (5 * n + 3)])
                pc = plsc.load_gather(pv, [base + (5 * n + 4)])
                pcx = (px + gif) / S
                pcy = (py + gjf) / S
                pw2 = pw * pw
                ph2 = ph * ph
                bx1 = pcx - pw2 * 0.5
                by1 = pcy - ph2 * 0.5
                bx2 = pcx + pw2 * 0.5
                by2 = pcy + ph2 * 0.5
                ix1 = jnp.maximum(bx1, x1)
                iy1 = jnp.maximum(by1, y1)
                ix2 = jnp.minimum(bx2, x2)
                iy2 = jnp.minimum(by2, y2)
                iw = jnp.maximum(ix2 - ix1, 0.0)
                ih = jnp.maximum(iy2 - iy1, 0.0)
                inter = iw * ih
                a1 = jnp.maximum(bx2 - bx1, 0.0) * jnp.maximum(by2 - by1, 0.0)
                a2 = jnp.maximum(x2 - x1, 0.0) * jnp.maximum(y2 - y1, 0.0)
                ious.append(inter / (a1 + a2 - inter + 1e-6))
                pb.append((px, py, pw, ph, pc))
            sel1 = ious[1] > ious[0]
            best = jnp.where(sel1, 1, 0)
            iou_b = jnp.where(sel1, ious[1], ious[0])
            pbx = jnp.where(sel1, pb[1][0], pb[0][0])
            pby = jnp.where(sel1, pb[1][1], pb[0][1])
            pbw = jnp.where(sel1, pb[1][2], pb[0][2])
            pbh = jnp.where(sel1, pb[1][3], pb[0][3])
            cp = jnp.where(sel1, pb[1][4], pb[0][4])

            tx = gcx * S - gif
            ty = gcy * S - gjf
            gww = jnp.maximum(gw, 1e-12)
            ghh = jnp.maximum(gh, 1e-12)
            tw = gww * _rsqrt_nr(gww)
            th = ghh * _rsqrt_nr(ghh)

            gdata.append(dict(valid=valid, cell=cell, base=base, lab=lab,
                              best=best, iou_b=iou_b, cp=cp,
                              coord=(pbx - tx) * (pbx - tx) +
                                    (pby - ty) * (pby - ty) +
                                    (pbw - tw) * (pbw - tw) +
                                    (pbh - th) * (pbh - th)))

        def dedup(keys):
            k0 = keys[0]
            k1 = jnp.where(gdata[1]["valid"], keys[1], -(iota + 200))
            kb[pl.ds(0, 16)] = k0
            kb[pl.ds(16, 16)] = k1
            al0 = iota < 16
            for sft in range(1, 24):
                al0 = jnp.logical_and(al0, k0 != kb[pl.ds(sft, 16)])
            al1 = iota < 16
            for sft in range(1, 8):
                al1 = jnp.logical_and(al1, k1 != kb[pl.ds(16 + sft, 16)])
            return [al0, al1]

        aliveA = dedup([d["cell"] * 2 + d["best"] for d in gdata])
        aliveB = dedup([d["cell"] * 128 + d["lab"] for d in gdata])
        aliveC = dedup([d["cell"] for d in gdata])

        for g in range(2):
            d = gdata[g]
            vA = jnp.where(jnp.logical_and(aliveA[g], d["valid"]), 1.0, 0.0)
            vB = jnp.where(jnp.logical_and(aliveB[g], d["valid"]), 1.0, 0.0)
            vC = jnp.where(jnp.logical_and(aliveC[g], d["valid"]), 1.0, 0.0)
            a_coord = a_coord + vA * d["coord"]
            dco = d["cp"] - d["iou_b"]
            a_obj = a_obj + vA * dco * dco
            a_mconf = a_mconf + vA * d["cp"] * d["cp"]
            plab = plsc.load_gather(pv, [d["base"] + (NB * 5) + d["lab"]])
            a_cls = a_cls + vB * (1.0 - 2.0 * plab)
            ssq = zf
            for c in range(C):
                pcl = plsc.load_gather(pv, [d["base"] + (NB * 5 + c)])
                ssq = ssq + pcl * pcl
            a_cls = a_cls + vC * ssq

    ov[pl.ds(0, 16)] = a_coord * L_COORD
    ov[pl.ds(16, 16)] = a_obj
    ov[pl.ds(32, 16)] = (a_dense - a_mconf) * L_NOOBJ
    ov[pl.ds(48, 16)] = a_cls
    pltpu.sync_copy(ov, out_hbm.at[wid])


def kernel(preds, boxes, labels):
    pv = preds.reshape(NTILES * PBLK)
    bt = jnp.pad(jnp.transpose(boxes, (0, 2, 1)),
                 ((0, 0), (0, 0), (0, 32 - NGT))).reshape(BATCH, 128)
    lb = lax.bitcast_convert_type(
        jnp.pad(labels, ((0, 0), (0, 32 - NGT))), jnp.float32)
    bl = jnp.concatenate([bt, lb], axis=1)
    part = _yolo_sc(pv, bl)
    sums = jnp.sum(part.reshape(NTILES, 4, 16), axis=(0, 2))
    coord, obj, noobj, cls = sums[0], sums[1], sums[2], sums[3]
    total = coord + obj + noobj + cls
    return total, coord, obj, noobj, cls


# speedup vs baseline: 1.0784x; 1.0784x over previous
import functools

import jax
import jax.numpy as jnp
from jax import lax
from jax.experimental import pallas as pl
from jax.experimental.pallas import tpu as pltpu
from jax.experimental.pallas import tpu_sc as plsc

S = 7
NB = 2
C = 80
W = NB * 5 + C
NCELL = S * S
BATCH = 128
NGT = 24
L_COORD = 5.0
L_NOOBJ = 0.5

WP = 4416
BL = 160
NC, NS = 2, 16
NTILES = NC * NS
IPT = BATCH // NTILES


def _rsqrt_nr(a):
    i = plsc.bitcast(a, jnp.int32)
    y = plsc.bitcast(jnp.int32(0x5F3759DF) - lax.shift_right_arithmetic(i, 1),
                     jnp.float32)
    for _ in range(3):
        y = y * (1.5 - 0.5 * a * y * y)
    return y


@functools.partial(
    pl.kernel,
    out_type=jax.ShapeDtypeStruct((NTILES, 64), jnp.float32),
    mesh=plsc.VectorSubcoreMesh(core_axis_name="c", subcore_axis_name="s",
                                num_cores=NC, num_subcores=NS),
    scratch_types=(
        [pltpu.VMEM((WP,), jnp.float32) for _ in range(IPT)]
        + [pltpu.VMEM((BL,), jnp.float32) for _ in range(IPT)]
        + [pltpu.VMEM((64,), jnp.int32),
           pltpu.VMEM((64,), jnp.float32)]
        + [pltpu.SemaphoreType.DMA for _ in range(IPT)]
    ),
    compiler_params=pltpu.CompilerParams(needs_layout_passes=False),
)
def _yolo_sc(preds_hbm, bl_hbm, out_hbm, *refs):
    pvs = refs[0:IPT]
    bls = refs[IPT:2 * IPT]
    kb = refs[2 * IPT]
    ov = refs[2 * IPT + 1]
    sems = refs[2 * IPT + 2:]

    wid = lax.axis_index("s") * NC + lax.axis_index("c")
    iota = lax.broadcasted_iota(jnp.int32, (16,), 0)
    zf = jnp.zeros((16,), jnp.float32)

    copies = []
    for t in range(IPT):
        img = wid * IPT + t
        c1 = pltpu.async_copy(preds_hbm.at[img], pvs[t], sems[t])
        c2 = pltpu.async_copy(bl_hbm.at[img], bls[t], sems[t])
        copies.append((c1, c2))

    for wdx in range(4):
        kb[pl.ds(wdx * 16, 16)] = -(iota + (wdx * 16 + 100))

    a_coord = a_obj = a_mconf = a_cls = a_dense = zf

    for t in range(IPT):
        for c in copies[t]:
            c.wait()
        pv = pvs[t]
        bv = bls[t]

        for gs in range(7):
            slot = iota + gs * 16
            off = (slot // NB) * W + (slot % NB) * 5 + 4
            v = plsc.load_gather(pv, [off])
            if gs == 6:
                v = jnp.where(slot < NCELL * NB, v, 0.0)
            a_dense = a_dense + v * v

        gdata = []
        for g in range(2):
            o = g * 16
            x1 = bv[pl.ds(0 + o, 16)]
            y1 = bv[pl.ds(32 + o, 16)]
            x2 = bv[pl.ds(64 + o, 16)]
            y2 = bv[pl.ds(96 + o, 16)]
            lab = plsc.bitcast(bv[pl.ds(128 + o, 16)], jnp.int32)
            valid = (iota + o) < NGT
            gcx = (x1 + x2) * 0.5
            gcy = (y1 + y2) * 0.5
            gw = x2 - x1
            gh = y2 - y1
            gi = jnp.clip((gcx * S).astype(jnp.int32), 0, S - 1)
            gj = jnp.clip((gcy * S).astype(jnp.int32), 0, S - 1)
            gif = gi.astype(jnp.float32)
            gjf = gj.astype(jnp.float32)
            cell = gj * S + gi
            base = cell * W

            ious = []
            pb = []
            for n in range(NB):
                px = plsc.load_gather(pv, [base + (5 * n + 0)])
                py = plsc.load_gather(pv, [base + (5 * n + 1)])
                pw = plsc.load_gather(pv, [base + (5 * n + 2)])
                ph = plsc.load_gather(pv, [base + (5 * n + 3)])
                pc = plsc.load_gather(pv, [base + (5 * n + 4)])
                pcx = (px + gif) / S
                pcy = (py + gjf) / S
                pw2 = pw * pw
                ph2 = ph * ph
                bx1 = pcx - pw2 * 0.5
                by1 = pcy - ph2 * 0.5
                bx2 = pcx + pw2 * 0.5
                by2 = pcy + ph2 * 0.5
                ix1 = jnp.maximum(bx1, x1)
                iy1 = jnp.maximum(by1, y1)
                ix2 = jnp.minimum(bx2, x2)
                iy2 = jnp.minimum(by2, y2)
                iw = jnp.maximum(ix2 - ix1, 0.0)
                ih = jnp.maximum(iy2 - iy1, 0.0)
                inter = iw * ih
                a1 = jnp.maximum(bx2 - bx1, 0.0) * jnp.maximum(by2 - by1, 0.0)
                a2 = jnp.maximum(x2 - x1, 0.0) * jnp.maximum(y2 - y1, 0.0)
                ious.append(inter / (a1 + a2 - inter + 1e-6))
                pb.append((px, py, pw, ph, pc))
            sel1 = ious[1] > ious[0]
            best = jnp.where(sel1, 1, 0)
            iou_b = jnp.where(sel1, ious[1], ious[0])
            pbx = jnp.where(sel1, pb[1][0], pb[0][0])
            pby = jnp.where(sel1, pb[1][1], pb[0][1])
            pbw = jnp.where(sel1, pb[1][2], pb[0][2])
            pbh = jnp.where(sel1, pb[1][3], pb[0][3])
            cp = jnp.where(sel1, pb[1][4], pb[0][4])

            tx = gcx * S - gif
            ty = gcy * S - gjf
            gww = jnp.maximum(gw, 1e-12)
            ghh = jnp.maximum(gh, 1e-12)
            tw = gww * _rsqrt_nr(gww)
            th = ghh * _rsqrt_nr(ghh)

            gdata.append(dict(valid=valid, cell=cell, base=base, lab=lab,
                              best=best, iou_b=iou_b, cp=cp,
                              coord=(pbx - tx) * (pbx - tx) +
                                    (pby - ty) * (pby - ty) +
                                    (pbw - tw) * (pbw - tw) +
                                    (pbh - th) * (pbh - th)))

        def dedup(keys):
            k0 = keys[0]
            k1 = jnp.where(gdata[1]["valid"], keys[1], -(iota + 200))
            kb[pl.ds(0, 16)] = k0
            kb[pl.ds(16, 16)] = k1
            al0 = iota < 16
            for sft in range(1, 24):
                al0 = jnp.logical_and(al0, k0 != kb[pl.ds(sft, 16)])
            al1 = iota < 16
            for sft in range(1, 8):
                al1 = jnp.logical_and(al1, k1 != kb[pl.ds(16 + sft, 16)])
            return [al0, al1]

        aliveA = dedup([d["cell"] * 2 + d["best"] for d in gdata])
        aliveB = dedup([d["cell"] * 128 + d["lab"] for d in gdata])
        aliveC = dedup([d["cell"] for d in gdata])

        for g in range(2):
            d = gdata[g]
            vA = jnp.where(jnp.logical_and(aliveA[g], d["valid"]), 1.0, 0.0)
            vB = jnp.where(jnp.logical_and(aliveB[g], d["valid"]), 1.0, 0.0)
            vC = jnp.where(jnp.logical_and(aliveC[g], d["valid"]), 1.0, 0.0)
            a_coord = a_coord + vA * d["coord"]
            dco = d["cp"] - d["iou_b"]
            a_obj = a_obj + vA * dco * dco
            a_mconf = a_mconf + vA * d["cp"] * d["cp"]
            plab = plsc.load_gather(pv, [d["base"] + (NB * 5) + d["lab"]])
            a_cls = a_cls + vB * (1.0 - 2.0 * plab)
            ssq = zf
            for c in range(C):
                pcl = plsc.load_gather(pv, [d["base"] + (NB * 5 + c)])
                ssq = ssq + pcl * pcl
            a_cls = a_cls + vC * ssq

    ov[pl.ds(0, 16)] = a_coord * L_COORD
    ov[pl.ds(16, 16)] = a_obj
    ov[pl.ds(32, 16)] = (a_dense - a_mconf) * L_NOOBJ
    ov[pl.ds(48, 16)] = a_cls
    pltpu.sync_copy(ov, out_hbm.at[wid])


def kernel(preds, boxes, labels):
    pv = jnp.pad(preds.reshape(BATCH, NCELL * W), ((0, 0), (0, WP - NCELL * W)))
    bt = jnp.pad(jnp.transpose(boxes, (0, 2, 1)),
                 ((0, 0), (0, 0), (0, 32 - NGT))).reshape(BATCH, 128)
    lb = lax.bitcast_convert_type(
        jnp.pad(labels, ((0, 0), (0, 32 - NGT))), jnp.float32)
    bl = jnp.concatenate([bt, lb], axis=1)
    part = _yolo_sc(pv, bl)
    sums = jnp.sum(part.reshape(NTILES, 4, 16), axis=(0, 2))
    coord, obj, noobj, cls = sums[0], sums[1], sums[2], sums[3]
    total = coord + obj + noobj + cls
    return total, coord, obj, noobj, cls
